# Initial kernel scaffold; baseline (speedup 1.0000x reference)
import jax
import jax.numpy as jnp
from jax.experimental import pallas as pl


def kernel(pred, target, skeleton_mask):
    raise NotImplementedError("write your pallas kernel here")


# speedup vs baseline: 1.1237x; 1.1237x over previous
import jax
import jax.numpy as jnp
from jax.experimental import pallas as pl
from jax.experimental.pallas import tpu as pltpu

W_POINT, W_CHAMFER, W_STRUCTURE = 1.0, 5.0, 2.0
BIG = 1e10


def _loss_kernel(pxr, pyr, pxc, pyc, pvc, txr, tyr, tvr, s0r, s1r,
                 out_total, out_point, out_chamfer, acc):
    b = pl.program_id(0)
    nb = pl.num_programs(0)

    @pl.when(b == 0)
    def _init():
        acc[0] = 0.0
        acc[1] = 0.0
        acc[2] = 0.0
        acc[3] = 0.0

    px_r = pxr[0]
    py_r = pyr[0]
    tx_r = txr[0]
    ty_r = tyr[0]
    tv_r = tvr[0]

    v = (tv_r == 1.0).astype(jnp.float32)
    ex = px_r - tx_r
    ey = py_r - ty_r
    err2 = ex * ex + ey * ey
    point_sse = jnp.sum(v * err2)
    smask = jnp.clip(s0r[0] + s1r[0], 0.0, 1.0)
    tmask = smask * v
    struct_sse = jnp.sum(tmask * err2)
    tm_sum = jnp.sum(tmask)

    px_c = pxc[0]
    py_c = pyc[0]
    pv_c = pvc[0]
    pm_c = (pv_c == 1.0).astype(jnp.float32)
    cnt_p = jnp.sum(pm_c)
    cnt_t = jnp.sum(v)

    a2_c = px_c * px_c + py_c * py_c
    b2_r = tx_r * tx_r + ty_r * ty_r
    b2t_r = b2_r + (1.0 - v) * BIG
    a2p_c = a2_c + (1.0 - pm_c) * BIG

    pxb = px_c.astype(jnp.bfloat16).astype(jnp.float32)
    pyb = py_c.astype(jnp.bfloat16).astype(jnp.float32)
    txb = tx_r.astype(jnp.bfloat16).astype(jnp.float32)
    tyb = ty_r.astype(jnp.bfloat16).astype(jnp.float32)
    cross = (-2.0 * pxb) * txb + (-2.0 * pyb) * tyb

    rowmin = jnp.min(cross + b2t_r, axis=1, keepdims=True)
    d2row = jnp.maximum(a2_c + rowmin, 0.0) + 1e-12
    min_dist_pred = jnp.sqrt(d2row)
    mean_p = jnp.sum(pm_c * min_dist_pred) / jnp.maximum(cnt_p, 1.0)

    colmin = jnp.min(cross + a2p_c, axis=0, keepdims=True)
    d2col = jnp.maximum(b2_r + colmin, 0.0) + 1e-12
    min_dist_tgt = jnp.sqrt(d2col)
    mean_t = jnp.sum(v * min_dist_tgt) / jnp.maximum(cnt_t, 1.0)

    valid_b = ((cnt_p > 0.0) & (cnt_t > 0.0)).astype(jnp.float32)
    chamfer = valid_b * (mean_p + mean_t) * 0.5

    acc[0] = acc[0] + point_sse
    acc[1] = acc[1] + struct_sse
    acc[2] = acc[2] + tm_sum
    acc[3] = acc[3] + chamfer

    @pl.when(b == nb - 1)
    def _finalize():
        n_elems = jnp.float32(nb) * jnp.float32(2 * pxr.shape[2])
        loss_point = acc[0] / n_elems
        loss_structure = jnp.where(acc[2] == 0.0, 0.0, acc[1] / n_elems)
        loss_chamfer = acc[3] / jnp.float32(nb)
        out_point[0, 0] = loss_point
        out_chamfer[0, 0] = loss_chamfer
        out_total[0, 0] = (W_POINT * loss_point + W_CHAMFER * loss_chamfer
                           + W_STRUCTURE * loss_structure)


def kernel(pred, target, skeleton_mask):
    B, N, _ = pred.shape
    f32 = jnp.float32

    pxr = pred[:, :, 0].reshape(B, 1, N)
    pyr = pred[:, :, 1].reshape(B, 1, N)
    pxc = pred[:, :, 0].reshape(B, N, 1)
    pyc = pred[:, :, 1].reshape(B, N, 1)
    pvc = pred[:, :, 2].reshape(B, N, 1)
    txr = target[:, :, 0].reshape(B, 1, N)
    tyr = target[:, :, 1].reshape(B, 1, N)
    tvr = target[:, :, 2].reshape(B, 1, N)
    s0r = skeleton_mask[:, :, 0].astype(f32).reshape(B, 1, N)
    s1r = skeleton_mask[:, :, 1].astype(f32).reshape(B, 1, N)

    row_spec = pl.BlockSpec((1, 1, N), lambda b: (b, 0, 0))
    col_spec = pl.BlockSpec((1, N, 1), lambda b: (b, 0, 0))
    out_spec = pl.BlockSpec(memory_space=pltpu.SMEM)

    out_shape = [jax.ShapeDtypeStruct((1, 1), f32)] * 3
    total, point, chamfer = pl.pallas_call(
        _loss_kernel,
        grid=(B,),
        in_specs=[row_spec, row_spec, col_spec, col_spec, col_spec,
                  row_spec, row_spec, row_spec, row_spec, row_spec],
        out_specs=[out_spec, out_spec, out_spec],
        out_shape=out_shape,
        scratch_shapes=[pltpu.SMEM((4,), f32)],
    )(pxr, pyr, pxc, pyc, pvc, txr, tyr, tvr, s0r, s1r)

    return (total[0, 0], point[0, 0], jnp.zeros((), f32), chamfer[0, 0])
